# baseline (speedup 1.0000x reference)
import functools

import jax
import jax.numpy as jnp
from jax.experimental import pallas as pl
from jax.experimental.pallas import tpu as pltpu
from jax.experimental.pallas import tpu_sc as plsc


_CHUNK = 128
_NC, _NS = 2, 16
_NW = _NC * _NS


def _transpose_body(tin_ref, out_ref):
    tin = tin_ref[...]
    e = tin.shape[0]
    ident = (jax.lax.broadcasted_iota(jnp.int32, (e, e), 0)
             == jax.lax.broadcasted_iota(jnp.int32, (e, e), 1)
             ).astype(jnp.float32)
    t1 = jax.lax.dot_general(
        tin, ident, dimension_numbers=(((0,), (0,)), ((), ())),
        preferred_element_type=jnp.float32,
    )
    cb = t1.shape[0]
    out_ref[:, :64] = t1[: cb // 2]
    out_ref[:, 64:] = t1[cb // 2:]


def _tc_table_prep(tableT):
    e, v = tableT.shape
    cb = 4096
    ng = pl.cdiv(v, cb)
    return pl.pallas_call(
        _transpose_body,
        grid=(ng,),
        in_specs=[pl.BlockSpec((e, cb), lambda i: (0, i))],
        out_specs=pl.BlockSpec((cb // 2, 2 * e), lambda i: (i, 0)),
        out_shape=jax.ShapeDtypeStruct((ng * cb // 2, 2 * e), jnp.float32),
    )(tableT)


def _sc_gather(table, idsT):
    l, b = idsT.shape
    e = table.shape[1]
    n_units = l * (b // _CHUNK)
    u_per_w = n_units // _NW
    half = b // 2
    mesh = plsc.VectorSubcoreMesh(core_axis_name="c", subcore_axis_name="s")

    @functools.partial(
        pl.kernel,
        out_type=jax.ShapeDtypeStruct((l * half, 2 * e), table.dtype),
        mesh=mesh,
        compiler_params=pltpu.CompilerParams(use_tc_tiling_on_sc=False),
        scratch_types=[
            pltpu.VMEM((_CHUNK,), jnp.int32),
            pltpu.VMEM((_CHUNK, e), jnp.float32),
            pltpu.SemaphoreType.DMA,
        ],
    )
    def gather_kernel(tbl_hbm, ids_hbm, out_hbm, idx_v, rows_v, sem):
        wid = jax.lax.axis_index("s") * _NC + jax.lax.axis_index("c")

        @pl.loop(wid * u_per_w, (wid + 1) * u_per_w)
        def _(u):
            g = u // 32
            p = (u // 16) & 1
            mb = u & 15
            src = p * half + mb * _CHUNK
            pltpu.sync_copy(ids_hbm.at[g, pl.ds(src, _CHUNK)], idx_v)
            for j in range(_CHUNK // 16):
                sl = pl.ds(j * 16, 16)
                v = idx_v[sl]
                rem = jax.lax.bitwise_and(v, 4095)
                twice = rem + rem
                adj = jnp.where(rem < 2048, twice, twice - 4095)
                idx_v[sl] = v - rem + adj
            pltpu.async_copy(tbl_hbm.at[idx_v], rows_v, sem).wait()
            q0 = g * half + mb * _CHUNK
            pltpu.sync_copy(
                rows_v, out_hbm.at[pl.ds(q0, _CHUNK), pl.ds(64 * p, e)])

    return gather_kernel(table, idsT)


_LB = 8


def _consumer_body(emb_ref, wt_ref, bt_ref, wg_ref, bg_ref,
                   tagT_ref, tmplT_ref, acc_ref):
    g = pl.program_id(0)
    ng = pl.num_programs(0)
    blk = emb_ref[...]
    even = blk[:, :, :64]
    odd = blk[:, :, 64:]
    se = jnp.sum(even, axis=0)
    so = jnp.sum(odd, axis=0)

    @pl.when(g == 0)
    def _():
        acc_ref[:2048, :] = se
        acc_ref[2048:, :] = so

    @pl.when(g > 0)
    def _():
        acc_ref[:2048, :] += se
        acc_ref[2048:, :] += so

    wg = wg_ref[...]
    tag_even = jax.lax.dot_general(
        wg, even, dimension_numbers=(((0,), (2,)), ((), ())),
        preferred_element_type=jnp.float32,
    )
    tag_odd = jax.lax.dot_general(
        wg, odd, dimension_numbers=(((0,), (2,)), ((), ())),
        preferred_element_type=jnp.float32,
    )
    bg = bg_ref[...].reshape(-1, 1, 1)
    tagT_ref[:, :, :2048] = tag_even + bg
    tagT_ref[:, :, 2048:] = tag_odd + bg

    @pl.when(g == ng - 1)
    def _():
        pooled = acc_ref[...] * (1.0 / (ng * _LB))
        tmplT_ref[...] = jax.lax.dot_general(
            wt_ref[...], pooled, dimension_numbers=(((0,), (1,)), ((), ())),
            preferred_element_type=jnp.float32,
        ) + bt_ref[...]


def _tc_consume(emb3, W_template, b_template, W_tag, b_tag):
    l, half_b, e2 = emb3.shape
    b = 2 * half_b
    template = W_template.shape[1]
    tag = W_tag.shape[1]
    grid = (l // _LB,)
    out_shapes = (
        jax.ShapeDtypeStruct((tag, l, b), jnp.float32),
        jax.ShapeDtypeStruct((template, b), jnp.float32),
    )
    return pl.pallas_call(
        _consumer_body,
        grid=grid,
        in_specs=[
            pl.BlockSpec((_LB, half_b, e2), lambda i: (i, 0, 0)),
            pl.BlockSpec((e2 // 2, template), lambda i: (0, 0)),
            pl.BlockSpec((template, 1), lambda i: (0, 0)),
            pl.BlockSpec((e2 // 2, tag), lambda i: (0, 0)),
            pl.BlockSpec((tag, 1), lambda i: (0, 0)),
        ],
        out_specs=(
            pl.BlockSpec((tag, _LB, b), lambda i: (0, i, 0)),
            pl.BlockSpec((template, b), lambda i: (0, 0)),
        ),
        out_shape=out_shapes,
        scratch_shapes=[pltpu.VMEM((b, e2 // 2), jnp.float32)],
    )(emb3, W_template, b_template.reshape(-1, 1),
      W_tag, b_tag.reshape(-1, 1))


def kernel(input_ids, table, W_template, b_template, W_tag, b_tag):
    b, l = input_ids.shape
    v, e = table.shape
    idsT = input_ids.astype(jnp.int32).T

    table_rm = _tc_table_prep(table.T)
    table_lin = table_rm.reshape(-1, e)
    emb = _sc_gather(table_lin, idsT)
    emb3 = emb.reshape(l, b // 2, 2 * e)

    tagT, tmplT = _tc_consume(emb3, W_template, b_template, W_tag, b_tag)
    template_logits = tmplT.T
    tag_logits = jnp.transpose(tagT, (2, 1, 0))
    return (template_logits, tag_logits)


# speedup vs baseline: 1.2093x; 1.2093x over previous
import functools

import jax
import jax.numpy as jnp
from jax.experimental import pallas as pl
from jax.experimental.pallas import tpu as pltpu
from jax.experimental.pallas import tpu_sc as plsc


_CHUNK = 128
_NC, _NS = 2, 16
_NW = _NC * _NS


def _transpose_body(tin_ref, out_ref):
    tin = tin_ref[...]
    t1 = tin.T
    cb = t1.shape[0]
    out_ref[:, :64] = t1[: cb // 2]
    out_ref[:, 64:] = t1[cb // 2:]


def _tc_table_prep(tableT):
    e, v = tableT.shape
    cb = 4096
    ng = pl.cdiv(v, cb)
    return pl.pallas_call(
        _transpose_body,
        grid=(ng,),
        in_specs=[pl.BlockSpec((e, cb), lambda i: (0, i))],
        out_specs=pl.BlockSpec((cb // 2, 2 * e), lambda i: (i, 0)),
        out_shape=jax.ShapeDtypeStruct((ng * cb // 2, 2 * e), jnp.float32),
    )(tableT)


def _sc_gather(table, idsT):
    l, b = idsT.shape
    e = table.shape[1]
    n_units = l * (b // _CHUNK)
    u_per_w = n_units // _NW
    half = b // 2
    mesh = plsc.VectorSubcoreMesh(core_axis_name="c", subcore_axis_name="s")

    @functools.partial(
        pl.kernel,
        out_type=jax.ShapeDtypeStruct((l * half, 2 * e), table.dtype),
        mesh=mesh,
        compiler_params=pltpu.CompilerParams(use_tc_tiling_on_sc=False),
        scratch_types=[
            pltpu.VMEM((_CHUNK,), jnp.int32),
            pltpu.VMEM((_CHUNK, e), jnp.float32),
            pltpu.VMEM((_CHUNK,), jnp.int32),
            pltpu.VMEM((_CHUNK, e), jnp.float32),
            pltpu.SemaphoreType.DMA,
            pltpu.SemaphoreType.DMA,
            pltpu.SemaphoreType.DMA,
        ],
    )
    def gather_kernel(tbl_hbm, ids_hbm, out_hbm,
                      idx_a, rows_a, idx_b, rows_b, sem_a, sem_b, sem_w):
        wid = jax.lax.axis_index("s") * _NC + jax.lax.axis_index("c")
        base = wid * u_per_w
        last = n_units - 1

        def load_remap(u, idx_v):
            g = u // 32
            p = (u // 16) & 1
            mb = u & 15
            src = p * half + mb * _CHUNK
            pltpu.sync_copy(ids_hbm.at[g, pl.ds(src, _CHUNK)], idx_v)
            for j in range(_CHUNK // 16):
                sl = pl.ds(j * 16, 16)
                v = idx_v[sl]
                rem = jax.lax.bitwise_and(v, 4095)
                twice = rem + rem
                adj = jnp.where(rem < 2048, twice, twice - 4095)
                idx_v[sl] = v - rem + adj

        def out_slice(u):
            g = u // 32
            p = (u // 16) & 1
            mb = u & 15
            q0 = g * half + mb * _CHUNK
            return out_hbm.at[pl.ds(q0, _CHUNK), pl.ds(64 * p, e)]

        load_remap(base, idx_a)
        pltpu.async_copy(tbl_hbm.at[idx_a], rows_a, sem_a)
        load_remap(base + 1, idx_b)

        @pl.loop(0, u_per_w, step=2)
        def _(i):
            u = base + i
            pltpu.make_async_copy(tbl_hbm.at[idx_a], rows_a, sem_a).wait()
            pltpu.async_copy(tbl_hbm.at[idx_b], rows_b, sem_b)
            pltpu.async_copy(rows_a, out_slice(u), sem_w)
            u2 = jnp.minimum(u + 2, last)
            load_remap(u2, idx_a)
            pltpu.make_async_copy(rows_a, out_slice(u), sem_w).wait()
            pltpu.make_async_copy(tbl_hbm.at[idx_b], rows_b, sem_b).wait()
            pltpu.async_copy(tbl_hbm.at[idx_a], rows_a, sem_a)
            pltpu.async_copy(rows_b, out_slice(u + 1), sem_w)
            u3 = jnp.minimum(u + 3, last)
            load_remap(u3, idx_b)
            pltpu.make_async_copy(rows_b, out_slice(u + 1), sem_w).wait()

        pltpu.make_async_copy(tbl_hbm.at[idx_a], rows_a, sem_a).wait()

    return gather_kernel(table, idsT)


_LB = 8


def _consumer_body(emb_ref, wt_ref, bt_ref, wg_ref, bg_ref,
                   tagT_ref, tmplT_ref, acc_ref):
    g = pl.program_id(0)
    ng = pl.num_programs(0)
    blk = emb_ref[...]
    even = blk[:, :, :64]
    odd = blk[:, :, 64:]
    se = jnp.sum(even, axis=0)
    so = jnp.sum(odd, axis=0)

    @pl.when(g == 0)
    def _():
        acc_ref[:2048, :] = se
        acc_ref[2048:, :] = so

    @pl.when(g > 0)
    def _():
        acc_ref[:2048, :] += se
        acc_ref[2048:, :] += so

    wg = wg_ref[...]
    tag_even = jax.lax.dot_general(
        wg, even, dimension_numbers=(((0,), (2,)), ((), ())),
        preferred_element_type=jnp.float32,
    )
    tag_odd = jax.lax.dot_general(
        wg, odd, dimension_numbers=(((0,), (2,)), ((), ())),
        preferred_element_type=jnp.float32,
    )
    bg = bg_ref[...].reshape(-1, 1, 1)
    tagT_ref[:, :, :2048] = tag_even + bg
    tagT_ref[:, :, 2048:] = tag_odd + bg

    @pl.when(g == ng - 1)
    def _():
        pooled = acc_ref[...] * (1.0 / (ng * _LB))
        tmplT_ref[...] = jax.lax.dot_general(
            wt_ref[...], pooled, dimension_numbers=(((0,), (1,)), ((), ())),
            preferred_element_type=jnp.float32,
        ) + bt_ref[...]


def _tc_consume(emb3, W_template, b_template, W_tag, b_tag):
    l, half_b, e2 = emb3.shape
    b = 2 * half_b
    template = W_template.shape[1]
    tag = W_tag.shape[1]
    grid = (l // _LB,)
    out_shapes = (
        jax.ShapeDtypeStruct((tag, l, b), jnp.float32),
        jax.ShapeDtypeStruct((template, b), jnp.float32),
    )
    return pl.pallas_call(
        _consumer_body,
        grid=grid,
        in_specs=[
            pl.BlockSpec((_LB, half_b, e2), lambda i: (i, 0, 0)),
            pl.BlockSpec((e2 // 2, template), lambda i: (0, 0)),
            pl.BlockSpec((template, 1), lambda i: (0, 0)),
            pl.BlockSpec((e2 // 2, tag), lambda i: (0, 0)),
            pl.BlockSpec((tag, 1), lambda i: (0, 0)),
        ],
        out_specs=(
            pl.BlockSpec((tag, _LB, b), lambda i: (0, i, 0)),
            pl.BlockSpec((template, b), lambda i: (0, 0)),
        ),
        out_shape=out_shapes,
        scratch_shapes=[pltpu.VMEM((b, e2 // 2), jnp.float32)],
    )(emb3, W_template, b_template.reshape(-1, 1),
      W_tag, b_tag.reshape(-1, 1))


def kernel(input_ids, table, W_template, b_template, W_tag, b_tag):
    b, l = input_ids.shape
    v, e = table.shape
    idsT = input_ids.astype(jnp.int32).T

    table_rm = _tc_table_prep(table.T)
    table_lin = table_rm.reshape(-1, e)
    emb = _sc_gather(table_lin, idsT)
    emb3 = emb.reshape(l, b // 2, 2 * e)

    tagT, tmplT = _tc_consume(emb3, W_template, b_template, W_tag, b_tag)
    template_logits = tmplT.T
    tag_logits = jnp.transpose(tagT, (2, 1, 0))
    return (template_logits, tag_logits)


# speedup vs baseline: 1.2133x; 1.0033x over previous
import functools

import jax
import jax.numpy as jnp
from jax.experimental import pallas as pl
from jax.experimental.pallas import tpu as pltpu
from jax.experimental.pallas import tpu_sc as plsc


_CHUNK = 128
_NC, _NS = 2, 16
_NW = _NC * _NS


def _transpose_body(tin_ref, out_ref):
    tin = tin_ref[...]
    t1 = tin.T
    cb = t1.shape[0]
    out_ref[:, :64] = t1[: cb // 2]
    out_ref[:, 64:] = t1[cb // 2:]


def _tc_table_prep(tableT):
    e, v = tableT.shape
    cb = 4096
    ng = pl.cdiv(v, cb)
    return pl.pallas_call(
        _transpose_body,
        grid=(ng,),
        in_specs=[pl.BlockSpec((e, cb), lambda i: (0, i))],
        out_specs=pl.BlockSpec((cb // 2, 2 * e), lambda i: (i, 0)),
        out_shape=jax.ShapeDtypeStruct((ng * cb // 2, 2 * e), jnp.float32),
        compiler_params=pltpu.CompilerParams(
            dimension_semantics=("parallel",)),
    )(tableT)


def _sc_gather(table, idsT):
    l, b = idsT.shape
    e = table.shape[1]
    n_units = l * (b // _CHUNK)
    u_per_w = n_units // _NW
    half = b // 2
    mesh = plsc.VectorSubcoreMesh(core_axis_name="c", subcore_axis_name="s")

    @functools.partial(
        pl.kernel,
        out_type=jax.ShapeDtypeStruct((l * half, 2 * e), table.dtype),
        mesh=mesh,
        compiler_params=pltpu.CompilerParams(use_tc_tiling_on_sc=False),
        scratch_types=[
            pltpu.VMEM((_CHUNK,), jnp.int32),
            pltpu.VMEM((_CHUNK, e), jnp.float32),
            pltpu.VMEM((_CHUNK,), jnp.int32),
            pltpu.VMEM((_CHUNK, e), jnp.float32),
            pltpu.SemaphoreType.DMA,
            pltpu.SemaphoreType.DMA,
            pltpu.SemaphoreType.DMA,
        ],
    )
    def gather_kernel(tbl_hbm, ids_hbm, out_hbm,
                      idx_a, rows_a, idx_b, rows_b, sem_a, sem_b, sem_w):
        wid = jax.lax.axis_index("s") * _NC + jax.lax.axis_index("c")
        base = wid * u_per_w
        last = n_units - 1

        def load_remap(u, idx_v):
            g = u // 32
            p = (u // 16) & 1
            mb = u & 15
            src = p * half + mb * _CHUNK
            pltpu.sync_copy(ids_hbm.at[g, pl.ds(src, _CHUNK)], idx_v)
            for j in range(_CHUNK // 16):
                sl = pl.ds(j * 16, 16)
                v = idx_v[sl]
                rem = jax.lax.bitwise_and(v, 4095)
                twice = rem + rem
                adj = jnp.where(rem < 2048, twice, twice - 4095)
                idx_v[sl] = v - rem + adj

        def out_slice(u):
            g = u // 32
            p = (u // 16) & 1
            mb = u & 15
            q0 = g * half + mb * _CHUNK
            return out_hbm.at[pl.ds(q0, _CHUNK), pl.ds(64 * p, e)]

        load_remap(base, idx_a)
        pltpu.async_copy(tbl_hbm.at[idx_a], rows_a, sem_a)
        load_remap(base + 1, idx_b)

        @pl.loop(0, u_per_w, step=2)
        def _(i):
            u = base + i
            pltpu.make_async_copy(tbl_hbm.at[idx_a], rows_a, sem_a).wait()
            pltpu.async_copy(tbl_hbm.at[idx_b], rows_b, sem_b)
            pltpu.async_copy(rows_a, out_slice(u), sem_w)
            u2 = jnp.minimum(u + 2, last)
            load_remap(u2, idx_a)
            pltpu.make_async_copy(rows_a, out_slice(u), sem_w).wait()
            pltpu.make_async_copy(tbl_hbm.at[idx_b], rows_b, sem_b).wait()
            pltpu.async_copy(tbl_hbm.at[idx_a], rows_a, sem_a)
            pltpu.async_copy(rows_b, out_slice(u + 1), sem_w)
            u3 = jnp.minimum(u + 3, last)
            load_remap(u3, idx_b)
            pltpu.make_async_copy(rows_b, out_slice(u + 1), sem_w).wait()

        pltpu.make_async_copy(tbl_hbm.at[idx_a], rows_a, sem_a).wait()

    return gather_kernel(table, idsT)


_LB = 8


def _consumer_body(emb_ref, wt_ref, bt_ref, wg_ref, bg_ref,
                   tagT_ref, tmplT_ref, acc_ref):
    g = pl.program_id(0)
    ng = pl.num_programs(0)
    blk = emb_ref[...]
    even = blk[:, :, :64]
    odd = blk[:, :, 64:]
    se = jnp.sum(even, axis=0)
    so = jnp.sum(odd, axis=0)

    @pl.when(g == 0)
    def _():
        acc_ref[:2048, :] = se
        acc_ref[2048:, :] = so

    @pl.when(g > 0)
    def _():
        acc_ref[:2048, :] += se
        acc_ref[2048:, :] += so

    wg = wg_ref[...]
    tag_even = jax.lax.dot_general(
        wg, even, dimension_numbers=(((0,), (2,)), ((), ())),
        preferred_element_type=jnp.float32,
    )
    tag_odd = jax.lax.dot_general(
        wg, odd, dimension_numbers=(((0,), (2,)), ((), ())),
        preferred_element_type=jnp.float32,
    )
    bg = bg_ref[...].reshape(-1, 1, 1)
    tagT_ref[:, :, :2048] = tag_even + bg
    tagT_ref[:, :, 2048:] = tag_odd + bg

    @pl.when(g == ng - 1)
    def _():
        pooled = acc_ref[...] * (1.0 / (ng * _LB))
        tmplT_ref[...] = jax.lax.dot_general(
            wt_ref[...], pooled, dimension_numbers=(((0,), (1,)), ((), ())),
            preferred_element_type=jnp.float32,
        ) + bt_ref[...]


def _tc_consume(emb3, W_template, b_template, W_tag, b_tag):
    l, half_b, e2 = emb3.shape
    b = 2 * half_b
    template = W_template.shape[1]
    tag = W_tag.shape[1]
    grid = (l // _LB,)
    out_shapes = (
        jax.ShapeDtypeStruct((tag, l, b), jnp.float32),
        jax.ShapeDtypeStruct((template, b), jnp.float32),
    )
    return pl.pallas_call(
        _consumer_body,
        grid=grid,
        in_specs=[
            pl.BlockSpec((_LB, half_b, e2), lambda i: (i, 0, 0)),
            pl.BlockSpec((e2 // 2, template), lambda i: (0, 0)),
            pl.BlockSpec((template, 1), lambda i: (0, 0)),
            pl.BlockSpec((e2 // 2, tag), lambda i: (0, 0)),
            pl.BlockSpec((tag, 1), lambda i: (0, 0)),
        ],
        out_specs=(
            pl.BlockSpec((tag, _LB, b), lambda i: (0, i, 0)),
            pl.BlockSpec((template, b), lambda i: (0, 0)),
        ),
        out_shape=out_shapes,
        scratch_shapes=[pltpu.VMEM((b, e2 // 2), jnp.float32)],
    )(emb3, W_template, b_template.reshape(-1, 1),
      W_tag, b_tag.reshape(-1, 1))


def kernel(input_ids, table, W_template, b_template, W_tag, b_tag):
    b, l = input_ids.shape
    v, e = table.shape
    idsT = input_ids.astype(jnp.int32).T

    table_rm = _tc_table_prep(table.T)
    table_lin = table_rm.reshape(-1, e)
    emb = _sc_gather(table_lin, idsT)
    emb3 = emb.reshape(l, b // 2, 2 * e)

    tagT, tmplT = _tc_consume(emb3, W_template, b_template, W_tag, b_tag)
    template_logits = tmplT.T
    tag_logits = jnp.transpose(tagT, (2, 1, 0))
    return (template_logits, tag_logits)


# speedup vs baseline: 1.4068x; 1.1594x over previous
import functools

import jax
import jax.numpy as jnp
from jax.experimental import pallas as pl
from jax.experimental.pallas import tpu as pltpu
from jax.experimental.pallas import tpu_sc as plsc


_CHUNK = 128
_NC, _NS = 2, 16
_NW = _NC * _NS
_PAIR = 16384


def _transpose_body(tin_ref, out_ref):
    tin = tin_ref[...]
    t1 = tin.T
    cb = t1.shape[0]
    out_ref[:, :64] = t1[: cb // 2]
    out_ref[:, 64:] = t1[cb // 2:]


def _tc_table_prep(tableT):
    e, v = tableT.shape
    cb = _PAIR
    ng = pl.cdiv(v, cb)
    return pl.pallas_call(
        _transpose_body,
        grid=(ng,),
        in_specs=[pl.BlockSpec((e, cb), lambda i: (0, i))],
        out_specs=pl.BlockSpec((cb // 2, 2 * e), lambda i: (i, 0)),
        out_shape=jax.ShapeDtypeStruct((ng * cb // 2, 2 * e), jnp.float32),
        compiler_params=pltpu.CompilerParams(
            dimension_semantics=("parallel",)),
    )(tableT)


def _sc_gather(table, idsT):
    l, b = idsT.shape
    e = table.shape[1]
    n_units = l * (b // _CHUNK)
    u_per_w = n_units // _NW
    half = b // 2
    mesh = plsc.VectorSubcoreMesh(core_axis_name="c", subcore_axis_name="s")

    @functools.partial(
        pl.kernel,
        out_type=jax.ShapeDtypeStruct((l * half, 2 * e), table.dtype),
        mesh=mesh,
        compiler_params=pltpu.CompilerParams(use_tc_tiling_on_sc=False),
        scratch_types=[
            pltpu.VMEM((_CHUNK,), jnp.int32),
            pltpu.VMEM((_CHUNK, e), jnp.float32),
            pltpu.VMEM((_CHUNK,), jnp.int32),
            pltpu.VMEM((_CHUNK, e), jnp.float32),
            pltpu.SemaphoreType.DMA,
            pltpu.SemaphoreType.DMA,
            pltpu.SemaphoreType.DMA,
        ],
    )
    def gather_kernel(tbl_hbm, ids_hbm, out_hbm,
                      idx_a, rows_a, idx_b, rows_b, sem_a, sem_b, sem_w):
        wid = jax.lax.axis_index("s") * _NC + jax.lax.axis_index("c")
        base = wid * u_per_w
        last = n_units - 1

        def load_remap(u, idx_v):
            g = u // 32
            p = (u // 16) & 1
            mb = u & 15
            src = p * half + mb * _CHUNK
            pltpu.sync_copy(ids_hbm.at[g, pl.ds(src, _CHUNK)], idx_v)
            for j in range(_CHUNK // 16):
                sl = pl.ds(j * 16, 16)
                v = idx_v[sl]
                rem = jax.lax.bitwise_and(v, _PAIR - 1)
                twice = rem + rem
                adj = jnp.where(rem < _PAIR // 2, twice, twice - (_PAIR - 1))
                idx_v[sl] = v - rem + adj

        def out_slice(u):
            g = u // 32
            p = (u // 16) & 1
            mb = u & 15
            q0 = g * half + mb * _CHUNK
            return out_hbm.at[pl.ds(q0, _CHUNK), pl.ds(64 * p, e)]

        load_remap(base, idx_a)
        pltpu.async_copy(tbl_hbm.at[idx_a], rows_a, sem_a)
        load_remap(base + 1, idx_b)

        @pl.loop(0, u_per_w, step=2)
        def _(i):
            u = base + i
            pltpu.make_async_copy(tbl_hbm.at[idx_a], rows_a, sem_a).wait()
            pltpu.async_copy(tbl_hbm.at[idx_b], rows_b, sem_b)
            pltpu.async_copy(rows_a, out_slice(u), sem_w)
            u2 = jnp.minimum(u + 2, last)
            load_remap(u2, idx_a)
            pltpu.make_async_copy(rows_a, out_slice(u), sem_w).wait()
            pltpu.make_async_copy(tbl_hbm.at[idx_b], rows_b, sem_b).wait()
            pltpu.async_copy(tbl_hbm.at[idx_a], rows_a, sem_a)
            pltpu.async_copy(rows_b, out_slice(u + 1), sem_w)
            u3 = jnp.minimum(u + 3, last)
            load_remap(u3, idx_b)
            pltpu.make_async_copy(rows_b, out_slice(u + 1), sem_w).wait()

        pltpu.make_async_copy(tbl_hbm.at[idx_a], rows_a, sem_a).wait()

    return gather_kernel(table, idsT)


_LB = 8


def _consumer_body(emb_ref, wt_ref, bt_ref, wg_ref, bg_ref,
                   tagT_ref, tmplT_ref, acc_ref):
    g = pl.program_id(0)
    ng = pl.num_programs(0)
    blk = emb_ref[...]
    even = blk[:, :, :64]
    odd = blk[:, :, 64:]
    se = jnp.sum(even, axis=0)
    so = jnp.sum(odd, axis=0)

    @pl.when(g == 0)
    def _():
        acc_ref[:2048, :] = se
        acc_ref[2048:, :] = so

    @pl.when(g > 0)
    def _():
        acc_ref[:2048, :] += se
        acc_ref[2048:, :] += so

    wg = wg_ref[...]
    tag_even = jax.lax.dot_general(
        wg, even, dimension_numbers=(((0,), (2,)), ((), ())),
        preferred_element_type=jnp.float32,
    )
    tag_odd = jax.lax.dot_general(
        wg, odd, dimension_numbers=(((0,), (2,)), ((), ())),
        preferred_element_type=jnp.float32,
    )
    bg = bg_ref[...].reshape(-1, 1, 1)
    tagT_ref[:, :, :2048] = tag_even + bg
    tagT_ref[:, :, 2048:] = tag_odd + bg

    @pl.when(g == ng - 1)
    def _():
        pooled = acc_ref[...] * (1.0 / (ng * _LB))
        tmplT_ref[...] = jax.lax.dot_general(
            wt_ref[...], pooled, dimension_numbers=(((0,), (1,)), ((), ())),
            preferred_element_type=jnp.float32,
        ) + bt_ref[...]


def _tc_consume(emb3, W_template, b_template, W_tag, b_tag):
    l, half_b, e2 = emb3.shape
    b = 2 * half_b
    template = W_template.shape[1]
    tag = W_tag.shape[1]
    grid = (l // _LB,)
    out_shapes = (
        jax.ShapeDtypeStruct((tag, l, b), jnp.float32),
        jax.ShapeDtypeStruct((template, b), jnp.float32),
    )
    return pl.pallas_call(
        _consumer_body,
        grid=grid,
        in_specs=[
            pl.BlockSpec((_LB, half_b, e2), lambda i: (i, 0, 0)),
            pl.BlockSpec((e2 // 2, template), lambda i: (0, 0)),
            pl.BlockSpec((template, 1), lambda i: (0, 0)),
            pl.BlockSpec((e2 // 2, tag), lambda i: (0, 0)),
            pl.BlockSpec((tag, 1), lambda i: (0, 0)),
        ],
        out_specs=(
            pl.BlockSpec((tag, _LB, b), lambda i: (0, i, 0)),
            pl.BlockSpec((template, b), lambda i: (0, 0)),
        ),
        out_shape=out_shapes,
        scratch_shapes=[pltpu.VMEM((b, e2 // 2), jnp.float32)],
    )(emb3, W_template, b_template.reshape(-1, 1),
      W_tag, b_tag.reshape(-1, 1))


def kernel(input_ids, table, W_template, b_template, W_tag, b_tag):
    b, l = input_ids.shape
    v, e = table.shape
    idsT = input_ids.astype(jnp.int32).T

    table_rm = _tc_table_prep(table.T)
    table_lin = table_rm.reshape(-1, e)
    emb = _sc_gather(table_lin, idsT)
    emb3 = emb.reshape(l, b // 2, 2 * e)

    tagT, tmplT = _tc_consume(emb3, W_template, b_template, W_tag, b_tag)
    template_logits = tmplT.T
    tag_logits = jnp.transpose(tagT, (2, 1, 0))
    return (template_logits, tag_logits)


# speedup vs baseline: 1.4438x; 1.0263x over previous
import functools

import jax
import jax.numpy as jnp
from jax.experimental import pallas as pl
from jax.experimental.pallas import tpu as pltpu
from jax.experimental.pallas import tpu_sc as plsc


_CHUNK = 128
_NC, _NS = 2, 16
_NW = _NC * _NS
_PAIR = 32768


def _transpose_body(tin_ref, out_ref):
    tin = tin_ref[...]
    t1 = tin.T
    cb = t1.shape[0]
    out_ref[:, :64] = t1[: cb // 2]
    out_ref[:, 64:] = t1[cb // 2:]


def _tc_table_prep(tableT):
    e, v = tableT.shape
    cb = _PAIR
    ng = pl.cdiv(v, cb)
    return pl.pallas_call(
        _transpose_body,
        grid=(ng,),
        in_specs=[pl.BlockSpec((e, cb), lambda i: (0, i))],
        out_specs=pl.BlockSpec((cb // 2, 2 * e), lambda i: (i, 0)),
        out_shape=jax.ShapeDtypeStruct((ng * cb // 2, 2 * e), jnp.float32),
        compiler_params=pltpu.CompilerParams(
            dimension_semantics=("parallel",)),
    )(tableT)


def _sc_gather(table, idsT):
    l, b = idsT.shape
    e = table.shape[1]
    n_units = l * (b // _CHUNK)
    u_per_w = n_units // _NW
    half = b // 2
    mesh = plsc.VectorSubcoreMesh(core_axis_name="c", subcore_axis_name="s")

    @functools.partial(
        pl.kernel,
        out_type=jax.ShapeDtypeStruct((l * half, 2 * e), table.dtype),
        mesh=mesh,
        compiler_params=pltpu.CompilerParams(use_tc_tiling_on_sc=False),
        scratch_types=[
            pltpu.VMEM((_CHUNK,), jnp.int32),
            pltpu.VMEM((_CHUNK, e), jnp.float32),
            pltpu.VMEM((_CHUNK,), jnp.int32),
            pltpu.VMEM((_CHUNK, e), jnp.float32),
            pltpu.SemaphoreType.DMA,
            pltpu.SemaphoreType.DMA,
            pltpu.SemaphoreType.DMA,
        ],
    )
    def gather_kernel(tbl_hbm, ids_hbm, out_hbm,
                      idx_a, rows_a, idx_b, rows_b, sem_a, sem_b, sem_w):
        wid = jax.lax.axis_index("s") * _NC + jax.lax.axis_index("c")
        base = wid * u_per_w
        last = n_units - 1

        def load_remap(u, idx_v):
            g = u // 32
            p = (u // 16) & 1
            mb = u & 15
            src = p * half + mb * _CHUNK
            pltpu.sync_copy(ids_hbm.at[g, pl.ds(src, _CHUNK)], idx_v)
            for j in range(_CHUNK // 16):
                sl = pl.ds(j * 16, 16)
                v = idx_v[sl]
                rem = jax.lax.bitwise_and(v, _PAIR - 1)
                twice = rem + rem
                adj = jnp.where(rem < _PAIR // 2, twice, twice - (_PAIR - 1))
                idx_v[sl] = v - rem + adj

        def out_slice(u):
            g = u // 32
            p = (u // 16) & 1
            mb = u & 15
            q0 = g * half + mb * _CHUNK
            return out_hbm.at[pl.ds(q0, _CHUNK), pl.ds(64 * p, e)]

        load_remap(base, idx_a)
        pltpu.async_copy(tbl_hbm.at[idx_a], rows_a, sem_a)
        load_remap(base + 1, idx_b)

        @pl.loop(0, u_per_w, step=2)
        def _(i):
            u = base + i
            pltpu.make_async_copy(tbl_hbm.at[idx_a], rows_a, sem_a).wait()
            pltpu.async_copy(tbl_hbm.at[idx_b], rows_b, sem_b)
            pltpu.async_copy(rows_a, out_slice(u), sem_w)
            u2 = jnp.minimum(u + 2, last)
            load_remap(u2, idx_a)
            pltpu.make_async_copy(rows_a, out_slice(u), sem_w).wait()
            pltpu.make_async_copy(tbl_hbm.at[idx_b], rows_b, sem_b).wait()
            pltpu.async_copy(tbl_hbm.at[idx_a], rows_a, sem_a)
            pltpu.async_copy(rows_b, out_slice(u + 1), sem_w)
            u3 = jnp.minimum(u + 3, last)
            load_remap(u3, idx_b)
            pltpu.make_async_copy(rows_b, out_slice(u + 1), sem_w).wait()

        pltpu.make_async_copy(tbl_hbm.at[idx_a], rows_a, sem_a).wait()

    return gather_kernel(table, idsT)


_LB = 8


def _consumer_body(emb_ref, wt_ref, bt_ref, wg_ref, bg_ref,
                   tagT_ref, tmplT_ref, acc_ref):
    g = pl.program_id(0)
    ng = pl.num_programs(0)
    blk = emb_ref[...]
    even = blk[:, :, :64]
    odd = blk[:, :, 64:]
    se = jnp.sum(even, axis=0)
    so = jnp.sum(odd, axis=0)

    @pl.when(g == 0)
    def _():
        acc_ref[:2048, :] = se
        acc_ref[2048:, :] = so

    @pl.when(g > 0)
    def _():
        acc_ref[:2048, :] += se
        acc_ref[2048:, :] += so

    wg = wg_ref[...]
    tag_even = jax.lax.dot_general(
        wg, even, dimension_numbers=(((0,), (2,)), ((), ())),
        preferred_element_type=jnp.float32,
    )
    tag_odd = jax.lax.dot_general(
        wg, odd, dimension_numbers=(((0,), (2,)), ((), ())),
        preferred_element_type=jnp.float32,
    )
    bg = bg_ref[...].reshape(-1, 1, 1)
    tagT_ref[:, :, :2048] = tag_even + bg
    tagT_ref[:, :, 2048:] = tag_odd + bg

    @pl.when(g == ng - 1)
    def _():
        pooled = acc_ref[...] * (1.0 / (ng * _LB))
        tmplT_ref[...] = jax.lax.dot_general(
            wt_ref[...], pooled, dimension_numbers=(((0,), (1,)), ((), ())),
            preferred_element_type=jnp.float32,
        ) + bt_ref[...]


def _tc_consume(emb3, W_template, b_template, W_tag, b_tag):
    l, half_b, e2 = emb3.shape
    b = 2 * half_b
    template = W_template.shape[1]
    tag = W_tag.shape[1]
    grid = (l // _LB,)
    out_shapes = (
        jax.ShapeDtypeStruct((tag, l, b), jnp.float32),
        jax.ShapeDtypeStruct((template, b), jnp.float32),
    )
    return pl.pallas_call(
        _consumer_body,
        grid=grid,
        in_specs=[
            pl.BlockSpec((_LB, half_b, e2), lambda i: (i, 0, 0)),
            pl.BlockSpec((e2 // 2, template), lambda i: (0, 0)),
            pl.BlockSpec((template, 1), lambda i: (0, 0)),
            pl.BlockSpec((e2 // 2, tag), lambda i: (0, 0)),
            pl.BlockSpec((tag, 1), lambda i: (0, 0)),
        ],
        out_specs=(
            pl.BlockSpec((tag, _LB, b), lambda i: (0, i, 0)),
            pl.BlockSpec((template, b), lambda i: (0, 0)),
        ),
        out_shape=out_shapes,
        scratch_shapes=[pltpu.VMEM((b, e2 // 2), jnp.float32)],
    )(emb3, W_template, b_template.reshape(-1, 1),
      W_tag, b_tag.reshape(-1, 1))


def kernel(input_ids, table, W_template, b_template, W_tag, b_tag):
    b, l = input_ids.shape
    v, e = table.shape
    idsT = input_ids.astype(jnp.int32).T

    table_rm = _tc_table_prep(table.T)
    table_lin = table_rm.reshape(-1, e)
    emb = _sc_gather(table_lin, idsT)
    emb3 = emb.reshape(l, b // 2, 2 * e)

    tagT, tmplT = _tc_consume(emb3, W_template, b_template, W_tag, b_tag)
    template_logits = tmplT.T
    tag_logits = jnp.transpose(tagT, (2, 1, 0))
    return (template_logits, tag_logits)
